# baseline (speedup 1.0000x reference)
import functools

import jax
import jax.numpy as jnp
from jax import lax
from jax.experimental import pallas as pl
from jax.experimental.pallas import tpu as pltpu
from jax.experimental.pallas import tpu_sc as plsc

NC = 2
NS = 16
L = 16
NW = NC * NS
K = 128

_mesh = lambda: plsc.VectorSubcoreMesh(
    core_axis_name="c", subcore_axis_name="s", num_cores=NC, num_subcores=NS
)


def _zero_vmem_2d(ref, rows, width):
    z = jnp.zeros((L,), jnp.float32)

    def body(i, _):
        for j in range(width // L):
            ref[i, pl.ds(j * L, L)] = z
        return 0

    lax.fori_loop(0, rows, body, 0)


def _zero_vmem_1d(ref, n):
    z = jnp.zeros((L,), jnp.float32)

    def body(i, _):
        ref[pl.ds(i * L, L)] = z
        return 0

    lax.fori_loop(0, n // L, body, 0)


def _make_degree_kernel(E_pad, NP):
    e_per_tile = E_pad // NW
    n_chunks = e_per_tile // K
    npt = NP // NS

    @functools.partial(
        pl.kernel,
        out_type=jax.ShapeDtypeStruct((NC, 2, NP), jnp.float32),
        mesh=_mesh(),
        compiler_params=pltpu.CompilerParams(needs_layout_passes=False),
        scratch_types=[
            pltpu.VMEM_SHARED((NS, NP), jnp.float32),
            pltpu.VMEM_SHARED((NS, NP), jnp.float32),
            pltpu.VMEM((NP,), jnp.float32),
            pltpu.VMEM((NP,), jnp.float32),
            pltpu.VMEM((e_per_tile,), jnp.int32),
            pltpu.VMEM((e_per_tile,), jnp.int32),
            pltpu.VMEM((NS, npt), jnp.float32),
            pltpu.VMEM((npt,), jnp.float32),
        ],
    )
    def deg_kernel(src_hbm, dst_hbm, out_hbm, stage_o, stage_i, hist_o, hist_i,
                   sidx, didx, comb, res):
        c = lax.axis_index("c")
        s = lax.axis_index("s")
        g = c * NS + s

        pltpu.sync_copy(src_hbm.at[pl.ds(g * e_per_tile, e_per_tile)], sidx)
        pltpu.sync_copy(dst_hbm.at[pl.ds(g * e_per_tile, e_per_tile)], didx)
        _zero_vmem_1d(hist_o, NP)
        _zero_vmem_1d(hist_i, NP)
        one = jnp.ones((L,), jnp.float32)

        def chunk(i, _):
            base = i * K
            for u in range(K // L):
                plsc.addupdate_scatter(hist_o, [sidx[pl.ds(base + u * L, L)]], one)
                plsc.addupdate_scatter(hist_i, [didx[pl.ds(base + u * L, L)]], one)
            return 0

        lax.fori_loop(0, n_chunks, chunk, 0)

        pltpu.sync_copy(hist_o, stage_o.at[s])
        pltpu.sync_copy(hist_i, stage_i.at[s])
        plsc.subcore_barrier()
        base_n = s * npt
        for a, stage in ((0, stage_o), (1, stage_i)):
            pltpu.sync_copy(stage.at[:, pl.ds(base_n, npt)], comb)

            def red(j, _):
                acc = comb[0, pl.ds(j * L, L)]
                for i in range(1, NS):
                    acc = acc + comb[i, pl.ds(j * L, L)]
                res[pl.ds(j * L, L)] = acc
                return 0

            lax.fori_loop(0, npt // L, red, 0)
            pltpu.sync_copy(res, out_hbm.at[c, a, pl.ds(base_n, npt)])

    return deg_kernel


def _make_agg_kernel(NP, D, E_pad):
    e_per_tile = E_pad // NW
    n_chunks = e_per_tile // K
    NSB = 2
    SBC = n_chunks // NSB
    assert n_chunks % NSB == 0 and SBC % 2 == 0
    rpt = NP // NS
    ZR = 32
    assert rpt % ZR == 0 and rpt % 8 == 0

    @functools.partial(
        pl.kernel,
        out_type=jax.ShapeDtypeStruct((NC, NP, D), jnp.float32),
        mesh=_mesh(),
        scratch_types=[
            pltpu.VMEM_SHARED((NP, D), jnp.float32),
            pltpu.VMEM((SBC, K), jnp.int32),
            pltpu.VMEM((SBC, K), jnp.int32),
            pltpu.VMEM((2, K, D), jnp.float32),
            pltpu.VMEM((ZR, D), jnp.float32),
            pltpu.SemaphoreType.DMA,
        ],
    )
    def agg_kernel(h_hbm, src_hbm, dst_hbm, out_hbm, agg, sidx, didx, rows, zbuf,
                   gsem):
        c = lax.axis_index("c")
        s = lax.axis_index("s")
        g = c * NS + s

        _zero_vmem_2d(zbuf, ZR, D)
        row0 = s * rpt
        for r in range(rpt // ZR):
            pltpu.sync_copy(zbuf, agg.at[pl.ds(row0 + r * ZR, ZR)])
        plsc.subcore_barrier()

        def fetch(i, slot):
            pltpu.async_copy(h_hbm.at[sidx.at[i]], rows.at[slot], gsem)

        def drain_scatter(i, slot):
            pltpu.make_async_copy(h_hbm.at[sidx.at[i]], rows.at[slot],
                                  gsem).wait()
            pltpu.sync_copy(rows.at[slot], agg.at[didx.at[i]], add=True)

        for sb in range(NSB):
            crow = g * n_chunks + sb * SBC
            pltpu.sync_copy(src_hbm.at[pl.ds(crow, SBC)], sidx)
            pltpu.sync_copy(dst_hbm.at[pl.ds(crow, SBC)], didx)

            fetch(0, 0)

            def block(j, _):
                fetch(2 * j + 1, 1)
                drain_scatter(2 * j, 0)
                fetch(2 * j + 2, 0)
                drain_scatter(2 * j + 1, 1)
                return 0

            lax.fori_loop(0, SBC // 2 - 1, block, 0)
            fetch(SBC - 1, 1)
            drain_scatter(SBC - 2, 0)
            drain_scatter(SBC - 1, 1)

        plsc.subcore_barrier()
        pltpu.sync_copy(agg.at[pl.ds(row0, rpt)], out_hbm.at[c, pl.ds(row0, rpt)])

    return agg_kernel


def _tc_mm_scale(x, w, ns):
    def body(x_ref, w_ref, ns_ref, o_ref):
        o_ref[...] = (
            jnp.dot(x_ref[...], w_ref[...], preferred_element_type=jnp.float32)
            * ns_ref[...]
        )

    return pl.pallas_call(
        body,
        out_shape=jax.ShapeDtypeStruct((x.shape[0], w.shape[1]), jnp.float32),
    )(x, w, ns)


def _tc_layer_mid(aggp, nd, b1, w, ns):
    def body(ap_ref, nd_ref, b1_ref, w_ref, ns_ref, o_ref):
        a = ap_ref[0] + ap_ref[1]
        h = jnp.maximum(a * nd_ref[...] + b1_ref[...], 0.0)
        o_ref[...] = (
            jnp.dot(h, w_ref[...], preferred_element_type=jnp.float32) * ns_ref[...]
        )

    return pl.pallas_call(
        body,
        out_shape=jax.ShapeDtypeStruct((aggp.shape[1], w.shape[1]), jnp.float32),
    )(aggp, nd, b1, w, ns)


def _tc_final(aggp, nd, b2, D_out):
    N = nd.shape[0]

    def body(ap_ref, nd_ref, b2_ref, o_ref):
        o_ref[...] = (
            ap_ref[0, :N, :D_out] + ap_ref[1, :N, :D_out]
        ) * nd_ref[...] + b2_ref[...]

    return pl.pallas_call(
        body,
        out_shape=jax.ShapeDtypeStruct((N, D_out), jnp.float32),
    )(aggp, nd, b2)


def kernel(features, edge_index, W1, b1, W2, b2):
    N, D_in = features.shape
    E = edge_index.shape[1]
    D_hid = W1.shape[1]
    D_out = W2.shape[1]

    NP = ((N + NS * L - 1) // (NS * L)) * (NS * L)
    E_pad = ((E + NW * 2 * K - 1) // (NW * 2 * K)) * (NW * 2 * K)

    fill = jnp.full((E_pad - E,), NP - 1, jnp.int32)
    src = jnp.concatenate([edge_index[0].astype(jnp.int32), fill])
    dst = jnp.concatenate([edge_index[1].astype(jnp.int32), fill])
    src2 = src.reshape(E_pad // K, K)
    dst2 = dst.reshape(E_pad // K, K)
    xp = jnp.pad(features, ((0, NP - N), (0, 0)))

    degp = _make_degree_kernel(E_pad, NP)(src, dst)
    ns = lax.rsqrt(jnp.clip(degp[0, 0] + degp[1, 0], 1.0))[:, None]
    nd = lax.rsqrt(jnp.clip(degp[0, 1] + degp[1, 1], 1.0))[:, None]

    h1s = _tc_mm_scale(xp, W1, ns)
    aggp1 = _make_agg_kernel(NP, D_hid, E_pad)(h1s, src2, dst2)
    D2 = 128
    W2p = jnp.pad(W2, ((0, 0), (0, D2 - D_out)))
    h2s = _tc_layer_mid(aggp1, nd, b1, W2p, ns)
    aggp2 = _make_agg_kernel(NP, D2, E_pad)(h2s, src2, dst2)
    return _tc_final(aggp2, nd[:N], b2, D_out)
